# baseline (speedup 1.0000x reference)
import jax
import jax.numpy as jnp
from jax.experimental import pallas as pl
from jax.experimental.pallas import tpu as pltpu


def _body(pos_ref, c_ref, u_ref, o_ref, *, seq_block, seqlen):
    j = pl.program_id(1)
    base = j * seq_block
    p = pos_ref[0]
    o_ref[...] = c_ref[...]

    @pl.when(jnp.logical_and(p < base + seq_block, p + seqlen > base))
    def _patch():
        for r in range(seqlen):
            g = p + r

            @pl.when(jnp.logical_and(g >= base, g < base + seq_block))
            def _row():
                o_ref[0, g - base, :] = u_ref[0, r, :]


def kernel(cache, update, pos):
    b, h, s, d = cache.shape
    sl = update.shape[2]
    cache3 = cache.reshape(b * h, s, d)
    upd3 = update.reshape(b * h, sl, d)
    seq_block = 1024
    grid = (b * h, s // seq_block)

    import functools
    body = functools.partial(_body, seq_block=seq_block, seqlen=sl)

    grid_spec = pltpu.PrefetchScalarGridSpec(
        num_scalar_prefetch=1,
        grid=grid,
        in_specs=[
            pl.BlockSpec((1, seq_block, d), lambda i, j, p: (i, j, 0)),
            pl.BlockSpec((1, sl, d), lambda i, j, p: (i, 0, 0)),
        ],
        out_specs=pl.BlockSpec((1, seq_block, d), lambda i, j, p: (i, j, 0)),
    )
    out3 = pl.pallas_call(
        body,
        grid_spec=grid_spec,
        out_shape=jax.ShapeDtypeStruct((b * h, s, d), cache.dtype),
        compiler_params=pltpu.CompilerParams(
            dimension_semantics=("arbitrary", "arbitrary"),
        ),
    )(pos, cache3, upd3)
    return out3.reshape(b, h, s, d)
